# baseline (speedup 1.0000x reference)
import functools

import jax
import jax.numpy as jnp
from jax import lax
from jax.experimental import pallas as pl
from jax.experimental.pallas import tpu as pltpu
from jax.experimental.pallas import tpu_sc as plsc

DIM = 16
VOCAB_PAD = 48


def _lut_body(table_ref, pos_ref, gamma_ref, beta_ref, out_ref):
    emb = table_ref[...] + pos_ref[...]
    mean = jnp.mean(emb, axis=-1, keepdims=True)
    var = jnp.mean((emb - mean) * (emb - mean), axis=-1, keepdims=True)
    normed = (emb - mean) * lax.rsqrt(var + 1e-5)
    out_ref[...] = jax.nn.sigmoid(normed * gamma_ref[...] + beta_ref[...])


def _compute_lut(emb_table, gamma, beta):
    v = emb_table.shape[0]
    table_p = jnp.pad(emb_table, ((0, VOCAB_PAD - v), (0, 0)))
    pos = jnp.arange(DIM, dtype=jnp.float32).reshape(1, DIM)
    return pl.pallas_call(
        _lut_body,
        out_shape=jax.ShapeDtypeStruct((VOCAB_PAD, DIM), jnp.float32),
    )(table_p, pos, gamma.reshape(1, DIM), beta.reshape(1, DIM))


NC, NS = 2, 16
NW = NC * NS
CHUNK = 3200
NB = 2


def _make_gather(total_rows):
    b_per_w = total_rows // NW
    n_chunks = b_per_w // CHUNK
    mesh = plsc.VectorSubcoreMesh(core_axis_name="c", subcore_axis_name="s")

    @functools.partial(
        pl.kernel,
        mesh=mesh,
        out_type=jax.ShapeDtypeStruct((total_rows * DIM,), jnp.float32),
        scratch_types=[
            pltpu.VMEM((VOCAB_PAD * DIM,), jnp.float32),
            pltpu.VMEM((NB, CHUNK), jnp.int32),
            pltpu.VMEM((NB, CHUNK * DIM), jnp.float32),
        ]
        + [pltpu.SemaphoreType.DMA] * (2 * NB),
        compiler_params=pltpu.CompilerParams(
            use_tc_tiling_on_sc=False, needs_layout_passes=False
        ),
    )
    def gather(lut_hbm, idx_hbm, out_hbm, lut_v, idx_v, rows_v, *sems):
        si, so = sems[0:NB], sems[NB : 2 * NB]
        wid = lax.axis_index("s") * NC + lax.axis_index("c")
        base = wid * b_per_w
        pltpu.sync_copy(lut_hbm, lut_v)

        idx_d, o_d = {}, {}

        def fire_idx(c):
            b = c % NB
            idx_d[c] = pltpu.async_copy(
                idx_hbm.at[pl.ds(base + c * CHUNK, CHUNK)], idx_v.at[b], si[b]
            )

        def fire_out(c):
            b = c % NB
            o_d[c] = pltpu.async_copy(
                rows_v.at[b],
                out_hbm.at[pl.ds((base + c * CHUNK) * DIM, CHUNK * DIM)],
                so[b],
            )

        obase = lax.iota(jnp.int32, 16) * DIM
        fire_idx(0)
        for g in range(n_chunks):
            b = g % NB
            idx_d[g].wait()
            if g + 1 < n_chunks:
                fire_idx(g + 1)
            if g >= NB:
                o_d[g - NB].wait()
            idx_b = idx_v.at[b]
            rows_b = rows_v.at[b]

            def grp(i, carry):
                r0 = i * 16
                a = idx_b[pl.ds(r0, 16)] * DIM
                ob = obase + r0 * DIM
                for c in range(DIM):
                    v = plsc.load_gather(lut_v, [a + c])
                    plsc.store_scatter(rows_b, [ob + c], v)
                return carry

            lax.fori_loop(0, CHUNK // 16, grp, 0)
            fire_out(g)
        for c in range(max(0, n_chunks - NB), n_chunks):
            o_d[c].wait()

    return gather


def kernel(x, emb_table, gamma, beta):
    b, l = x.shape
    lut = _compute_lut(emb_table, gamma, beta)
    idx = x.reshape(-1).astype(jnp.int32)
    out = _make_gather(b * l)(lut.reshape(-1), idx)
    return out.reshape(b, l, DIM)


# speedup vs baseline: 1.1440x; 1.1440x over previous
import functools

import jax
import jax.numpy as jnp
from jax import lax
from jax.experimental import pallas as pl
from jax.experimental.pallas import tpu as pltpu
from jax.experimental.pallas import tpu_sc as plsc

DIM = 16
VOCAB_PAD = 48


def _lut_body(table_ref, pos_ref, gamma_ref, beta_ref, out_ref):
    emb = table_ref[...] + pos_ref[...]
    mean = jnp.mean(emb, axis=-1, keepdims=True)
    var = jnp.mean((emb - mean) * (emb - mean), axis=-1, keepdims=True)
    normed = (emb - mean) * lax.rsqrt(var + 1e-5)
    out_ref[...] = jax.nn.sigmoid(normed * gamma_ref[...] + beta_ref[...])


def _compute_lut(emb_table, gamma, beta):
    v = emb_table.shape[0]
    table_p = jnp.pad(emb_table, ((0, VOCAB_PAD - v), (0, 0)))
    pos = jnp.arange(DIM, dtype=jnp.float32).reshape(1, DIM)
    return pl.pallas_call(
        _lut_body,
        out_shape=jax.ShapeDtypeStruct((VOCAB_PAD, DIM), jnp.float32),
    )(table_p, pos, gamma.reshape(1, DIM), beta.reshape(1, DIM))


NC, NS = 2, 16
NW = NC * NS
CHUNK = 3200
NB = 2


def _make_gather(total_rows):
    b_per_w = total_rows // NW
    n_chunks = b_per_w // CHUNK
    mesh = plsc.VectorSubcoreMesh(core_axis_name="c", subcore_axis_name="s")

    @functools.partial(
        pl.kernel,
        mesh=mesh,
        out_type=jax.ShapeDtypeStruct((total_rows * DIM,), jnp.float32),
        scratch_types=[
            pltpu.VMEM((VOCAB_PAD * DIM,), jnp.float32),
            pltpu.VMEM((NB, CHUNK), jnp.int32),
            pltpu.VMEM((NB, CHUNK * DIM), jnp.float32),
        ]
        + [pltpu.SemaphoreType.DMA] * (2 * NB),
        compiler_params=pltpu.CompilerParams(
            use_tc_tiling_on_sc=False, needs_layout_passes=False
        ),
    )
    def gather(lut_hbm, idx_hbm, out_hbm, lut_v, idx_v, rows_v, *sems):
        si, so = sems[0:NB], sems[NB : 2 * NB]
        wid = lax.axis_index("s") * NC + lax.axis_index("c")
        base = wid * b_per_w
        pltpu.sync_copy(lut_hbm, lut_v)

        def idx_desc(c, b):
            return pltpu.make_async_copy(
                idx_hbm.at[pl.ds(base + c * CHUNK, CHUNK)], idx_v.at[b], si[b]
            )

        def out_desc(c, b):
            return pltpu.make_async_copy(
                rows_v.at[b],
                out_hbm.at[pl.ds((base + c * CHUNK) * DIM, CHUNK * DIM)],
                so[b],
            )

        obase = lax.iota(jnp.int32, 16) * DIM
        for b in range(NB):
            idx_desc(b, b).start()

        def pair(k, carry):
            for b in range(NB):
                g = k * NB + b
                idx_desc(g, b).wait()

                @pl.when(k > 0)
                def _drain_out():
                    out_desc(g - NB, b).wait()

                idx_b = idx_v.at[b]
                rows_b = rows_v.at[b]

                @plsc.parallel_loop(0, CHUNK, 16, unroll=2)
                def grp(r0):
                    a = idx_b[pl.ds(r0, 16)] * DIM
                    ob = obase + r0 * DIM
                    for c in range(DIM):
                        v = plsc.load_gather(lut_v, [a + c])
                        plsc.store_scatter(rows_b, [ob + c], v)

                out_desc(g, b).start()

                @pl.when(g + NB < n_chunks)
                def _prefetch():
                    idx_desc(g + NB, b).start()

            return carry

        lax.fori_loop(0, n_chunks // NB, pair, 0)
        for b in range(NB):
            out_desc(n_chunks - NB + b, b).wait()

    return gather


def kernel(x, emb_table, gamma, beta):
    b, l = x.shape
    lut = _compute_lut(emb_table, gamma, beta)
    idx = x.reshape(-1).astype(jnp.int32)
    out = _make_gather(b * l)(lut.reshape(-1), idx)
    return out.reshape(b, l, DIM)
